# baseline (speedup 1.0000x reference)
import functools

import jax
import jax.numpy as jnp
from jax import lax
from jax.experimental import pallas as pl
from jax.experimental.pallas import tpu as pltpu
from jax.experimental.pallas import tpu_sc as plsc

_N = 200000
_C = 64
_NW = 32
_CHUNK = 512
_NFULL = _N // _CHUNK
_TAIL = _N - _NFULL * _CHUNK
_ROUNDS = _NFULL // _NW
_EXTRA = _NFULL - _ROUNDS * _NW
_VPAD = _CHUNK + 128
_VTAIL = 384


def _sc_body(feats_hbm, velt_hbm, out_hbm, fbuf, vbuf):
    wid = lax.axis_index("c") * jnp.int32(16) + lax.axis_index("s")
    iota = lax.iota(jnp.int32, 16)
    dnums = lax.GatherDimensionNumbers(
        offset_dims=(), collapsed_slice_dims=(0,), start_index_map=(0,))

    def do_chunk(r0, nrows, nvel):
        pltpu.sync_copy(feats_hbm.at[pl.ds(r0, nrows), :],
                        fbuf.at[pl.ds(jnp.int32(0), nrows), :])
        pltpu.sync_copy(velt_hbm.at[:, pl.ds(r0, nvel)],
                        vbuf.at[:, pl.ds(jnp.int32(0), nvel)])

        def row_body(r, carry):
            f0 = fbuf[r, pl.ds(0, 16)]
            f1 = fbuf[r, pl.ds(16, 16)]
            f2 = fbuf[r, pl.ds(32, 16)]
            f3 = fbuf[r, pl.ds(48, 16)]
            acc = jnp.abs(f0) + jnp.abs(f1) + jnp.abs(f2) + jnp.abs(f3)
            for step in (8, 4, 2, 1):
                perm = jnp.bitwise_xor(iota, jnp.int32(step))
                acc = acc + lax.gather(
                    acc, perm[:, None], dnums, slice_sizes=(1,),
                    mode=lax.GatherScatterMode.PROMISE_IN_BOUNDS)
            base16 = pl.multiple_of(jnp.bitwise_and(r, jnp.int32(-16)), 16)
            lanevec = jnp.full((16,), jnp.bitwise_and(r, jnp.int32(15)),
                               jnp.int32)
            vxw = vbuf[0, pl.ds(base16, 16)]
            vyw = vbuf[1, pl.ds(base16, 16)]
            vx = lax.gather(vxw, lanevec[:, None], dnums, slice_sizes=(1,),
                            mode=lax.GatherScatterMode.PROMISE_IN_BOUNDS)
            vy = lax.gather(vyw, lanevec[:, None], dnums, slice_sizes=(1,),
                            mode=lax.GatherScatterMode.PROMISE_IN_BOUNDS)
            denom = (jnp.float32(1.0)
                     + jnp.float32(0.25) * (jnp.abs(vx) + jnp.abs(vy)))
            diff = acc / jnp.maximum(acc, jnp.float32(1e-6))
            scale = jnp.float32(1.0) - jnp.exp(-diff) / denom
            fbuf[r, pl.ds(0, 16)] = f0 * scale
            fbuf[r, pl.ds(16, 16)] = f1 * scale
            fbuf[r, pl.ds(32, 16)] = f2 * scale
            fbuf[r, pl.ds(48, 16)] = f3 * scale
            return carry

        lax.fori_loop(jnp.int32(0), jnp.int32(nrows), row_body, jnp.int32(0))
        pltpu.sync_copy(fbuf.at[pl.ds(jnp.int32(0), nrows), :],
                        out_hbm.at[pl.ds(r0, nrows), :])

    def round_body(k, carry):
        do_chunk((wid + k * jnp.int32(_NW)) * jnp.int32(_CHUNK), _CHUNK,
                 _CHUNK)
        return carry

    lax.fori_loop(jnp.int32(0), jnp.int32(_ROUNDS), round_body, jnp.int32(0))

    @pl.when(wid < jnp.int32(_EXTRA))
    def _():
        do_chunk((wid + jnp.int32(_ROUNDS * _NW)) * jnp.int32(_CHUNK), _CHUNK,
                 _CHUNK)

    @pl.when(wid == jnp.int32(_EXTRA))
    def _():
        do_chunk(jnp.int32(_NFULL * _CHUNK), _TAIL, _VTAIL)


def kernel(feats, vel_xy, coords):
    del coords
    n, c = feats.shape
    feats = feats.astype(jnp.float32)
    velt = vel_xy.astype(jnp.float32).T
    velt = jnp.pad(velt, ((0, 0), (0, _NFULL * _CHUNK + _VTAIL - n)))
    mesh = plsc.VectorSubcoreMesh(core_axis_name="c", subcore_axis_name="s")
    k = functools.partial(
        pl.kernel,
        mesh=mesh,
        out_type=jax.ShapeDtypeStruct((n, c), jnp.float32),
        scratch_types=[
            pltpu.VMEM((_CHUNK, _C), jnp.float32),
            pltpu.VMEM((2, _VPAD), jnp.float32),
        ],
    )(_sc_body)
    return k(feats, velt)


# speedup vs baseline: 1.6383x; 1.6383x over previous
import functools

import jax
import jax.numpy as jnp
from jax import lax
from jax.experimental import pallas as pl
from jax.experimental.pallas import tpu as pltpu
from jax.experimental.pallas import tpu_sc as plsc

_N = 200000
_C = 64
_NW = 32
_CHUNK = 512
_NFULL = _N // _CHUNK
_TAIL = _N - _NFULL * _CHUNK
_ROUNDS = _NFULL // _NW
_EXTRA = _NFULL - _ROUNDS * _NW
_VPAD = _CHUNK + 128
_VTAIL = 384


def _sc_body(feats_hbm, velt_hbm, out_hbm, fbuf, vbuf):
    wid = lax.axis_index("c") * jnp.int32(16) + lax.axis_index("s")
    iota = lax.iota(jnp.int32, 16)
    dnums = lax.GatherDimensionNumbers(
        offset_dims=(), collapsed_slice_dims=(0,), start_index_map=(0,))

    def do_chunk(r0, nrows, nvel):
        pltpu.sync_copy(feats_hbm.at[pl.ds(r0, nrows), :],
                        fbuf.at[pl.ds(jnp.int32(0), nrows), :])
        pltpu.sync_copy(velt_hbm.at[:, pl.ds(r0, nvel)],
                        vbuf.at[:, pl.ds(jnp.int32(0), nvel)])

        def group_body(g, carry):
            base = pl.multiple_of(g * jnp.int32(16), 16)
            s16 = jnp.zeros((16,), jnp.float32)
            for j in range(16):
                r = base + jnp.int32(j)
                acc = (jnp.abs(fbuf[r, pl.ds(0, 16)])
                       + jnp.abs(fbuf[r, pl.ds(16, 16)])
                       + jnp.abs(fbuf[r, pl.ds(32, 16)])
                       + jnp.abs(fbuf[r, pl.ds(48, 16)]))
                for step in (8, 4, 2, 1):
                    perm = jnp.bitwise_xor(iota, jnp.int32(step))
                    acc = acc + lax.gather(
                        acc, perm[:, None], dnums, slice_sizes=(1,),
                        mode=lax.GatherScatterMode.PROMISE_IN_BOUNDS)
                s16 = jnp.where(iota == jnp.int32(j), acc, s16)
            speed = jnp.abs(vbuf[0, pl.ds(base, 16)])
            speed = speed + jnp.abs(vbuf[1, pl.ds(base, 16)])
            diff = s16 / jnp.maximum(s16, jnp.float32(1e-6))
            gate = jnp.exp(-diff) / (jnp.float32(1.0)
                                     + jnp.float32(0.25) * speed)
            scale16 = jnp.float32(1.0) - gate
            for j in range(16):
                r = base + jnp.int32(j)
                lane = jnp.full((16,), j, jnp.int32)
                sc = lax.gather(scale16, lane[:, None], dnums,
                                slice_sizes=(1,),
                                mode=lax.GatherScatterMode.PROMISE_IN_BOUNDS)
                fbuf[r, pl.ds(0, 16)] = fbuf[r, pl.ds(0, 16)] * sc
                fbuf[r, pl.ds(16, 16)] = fbuf[r, pl.ds(16, 16)] * sc
                fbuf[r, pl.ds(32, 16)] = fbuf[r, pl.ds(32, 16)] * sc
                fbuf[r, pl.ds(48, 16)] = fbuf[r, pl.ds(48, 16)] * sc
            return carry

        lax.fori_loop(jnp.int32(0), jnp.int32(nrows // 16), group_body,
                      jnp.int32(0))
        pltpu.sync_copy(fbuf.at[pl.ds(jnp.int32(0), nrows), :],
                        out_hbm.at[pl.ds(r0, nrows), :])

    def round_body(k, carry):
        do_chunk((wid + k * jnp.int32(_NW)) * jnp.int32(_CHUNK), _CHUNK,
                 _CHUNK)
        return carry

    lax.fori_loop(jnp.int32(0), jnp.int32(_ROUNDS), round_body, jnp.int32(0))

    @pl.when(wid < jnp.int32(_EXTRA))
    def _():
        do_chunk((wid + jnp.int32(_ROUNDS * _NW)) * jnp.int32(_CHUNK), _CHUNK,
                 _CHUNK)

    @pl.when(wid == jnp.int32(_EXTRA))
    def _():
        do_chunk(jnp.int32(_NFULL * _CHUNK), _TAIL, _VTAIL)


def kernel(feats, vel_xy, coords):
    del coords
    n, c = feats.shape
    feats = feats.astype(jnp.float32)
    velt = vel_xy.astype(jnp.float32).T
    velt = jnp.pad(velt, ((0, 0), (0, _NFULL * _CHUNK + _VTAIL - n)))
    mesh = plsc.VectorSubcoreMesh(core_axis_name="c", subcore_axis_name="s")
    k = functools.partial(
        pl.kernel,
        mesh=mesh,
        out_type=jax.ShapeDtypeStruct((n, c), jnp.float32),
        scratch_types=[
            pltpu.VMEM((_CHUNK, _C), jnp.float32),
            pltpu.VMEM((2, _VPAD), jnp.float32),
        ],
    )(_sc_body)
    return k(feats, velt)


# speedup vs baseline: 1.9728x; 1.2042x over previous
import functools

import jax
import jax.numpy as jnp
from jax import lax
from jax.experimental import pallas as pl
from jax.experimental.pallas import tpu as pltpu
from jax.experimental.pallas import tpu_sc as plsc

_N = 200000
_C = 64
_NW = 32
_CHUNK = 128
_NFULL = _N // _CHUNK
_TAIL = _N - _NFULL * _CHUNK
_ROUNDS = _NFULL // _NW
_EXTRA = _NFULL - _ROUNDS * _NW
_NPAIRS = _ROUNDS // 2
_VPAD = 2 * _CHUNK
_VTAIL = 128


def _sc_body(feats_hbm, velt_hbm, out_hbm,
             f0, f1, o0, o1, v0, v1, sin0, sin1, sout0, sout1):
    wid = lax.axis_index("c") * jnp.int32(16) + lax.axis_index("s")
    iota = lax.iota(jnp.int32, 16)
    dnums = lax.GatherDimensionNumbers(
        offset_dims=(), collapsed_slice_dims=(0,), start_index_map=(0,))

    def in_descs(ci, fb, vb, sem):
        r0 = ci * jnp.int32(_CHUNK)
        return (
            pltpu.make_async_copy(feats_hbm.at[pl.ds(r0, _CHUNK), :], fb,
                                  sem),
            pltpu.make_async_copy(velt_hbm.at[:, pl.ds(r0, _CHUNK)],
                                  vb.at[:, pl.ds(jnp.int32(0), _CHUNK)], sem),
        )

    def out_desc(ci, ob, sem):
        r0 = ci * jnp.int32(_CHUNK)
        return pltpu.make_async_copy(ob, out_hbm.at[pl.ds(r0, _CHUNK), :],
                                     sem)

    def start_in(ci, fb, vb, sem):
        for d in in_descs(ci, fb, vb, sem):
            d.start()

    def wait_in(ci, fb, vb, sem):
        for d in in_descs(ci, fb, vb, sem):
            d.wait()

    def compute(fb, vb, ob, ngroups):
        def group_body(g, carry):
            base = pl.multiple_of(g * jnp.int32(16), 16)
            s16 = jnp.zeros((16,), jnp.float32)
            for j in range(16):
                r = base + jnp.int32(j)
                acc = (jnp.abs(fb[r, pl.ds(0, 16)])
                       + jnp.abs(fb[r, pl.ds(16, 16)])
                       + jnp.abs(fb[r, pl.ds(32, 16)])
                       + jnp.abs(fb[r, pl.ds(48, 16)]))
                for step in (8, 4, 2, 1):
                    perm = jnp.bitwise_xor(iota, jnp.int32(step))
                    acc = acc + lax.gather(
                        acc, perm[:, None], dnums, slice_sizes=(1,),
                        mode=lax.GatherScatterMode.PROMISE_IN_BOUNDS)
                s16 = jnp.where(iota == jnp.int32(j), acc, s16)
            speed = jnp.abs(vb[0, pl.ds(base, 16)])
            speed = speed + jnp.abs(vb[1, pl.ds(base, 16)])
            diff = s16 / jnp.maximum(s16, jnp.float32(1e-6))
            gate = jnp.exp(-diff) / (jnp.float32(1.0)
                                     + jnp.float32(0.25) * speed)
            scale16 = jnp.float32(1.0) - gate
            for j in range(16):
                r = base + jnp.int32(j)
                lane = jnp.full((16,), j, jnp.int32)
                sc = lax.gather(scale16, lane[:, None], dnums,
                                slice_sizes=(1,),
                                mode=lax.GatherScatterMode.PROMISE_IN_BOUNDS)
                ob[r, pl.ds(0, 16)] = fb[r, pl.ds(0, 16)] * sc
                ob[r, pl.ds(16, 16)] = fb[r, pl.ds(16, 16)] * sc
                ob[r, pl.ds(32, 16)] = fb[r, pl.ds(32, 16)] * sc
                ob[r, pl.ds(48, 16)] = fb[r, pl.ds(48, 16)] * sc
            return carry

        lax.fori_loop(jnp.int32(0), jnp.int32(ngroups), group_body,
                      jnp.int32(0))

    def ci_of(k):
        return wid + k * jnp.int32(_NW)

    start_in(ci_of(jnp.int32(0)), f0, v0, sin0)
    start_in(ci_of(jnp.int32(1)), f1, v1, sin1)

    def pair_body(p, carry):
        a = jnp.int32(2) * p
        b = a + jnp.int32(1)
        wait_in(ci_of(a), f0, v0, sin0)

        @pl.when(p > jnp.int32(0))
        def _():
            out_desc(ci_of(a - jnp.int32(2)), o0, sout0).wait()

        compute(f0, v0, o0, _CHUNK // 16)

        @pl.when(p < jnp.int32(_NPAIRS - 1))
        def _():
            start_in(ci_of(a + jnp.int32(2)), f0, v0, sin0)

        out_desc(ci_of(a), o0, sout0).start()

        wait_in(ci_of(b), f1, v1, sin1)

        @pl.when(p > jnp.int32(0))
        def _():
            out_desc(ci_of(b - jnp.int32(2)), o1, sout1).wait()

        compute(f1, v1, o1, _CHUNK // 16)

        @pl.when(p < jnp.int32(_NPAIRS - 1))
        def _():
            start_in(ci_of(b + jnp.int32(2)), f1, v1, sin1)

        out_desc(ci_of(b), o1, sout1).start()
        return carry

    lax.fori_loop(jnp.int32(0), jnp.int32(_NPAIRS), pair_body, jnp.int32(0))
    out_desc(ci_of(jnp.int32(_ROUNDS - 2)), o0, sout0).wait()
    out_desc(ci_of(jnp.int32(_ROUNDS - 1)), o1, sout1).wait()

    @pl.when(wid < jnp.int32(_EXTRA))
    def _():
        ci = wid + jnp.int32(_ROUNDS * _NW)
        r0 = ci * jnp.int32(_CHUNK)
        pltpu.sync_copy(feats_hbm.at[pl.ds(r0, _CHUNK), :], f0)
        pltpu.sync_copy(velt_hbm.at[:, pl.ds(r0, _CHUNK)],
                        v0.at[:, pl.ds(jnp.int32(0), _CHUNK)])
        compute(f0, v0, o0, _CHUNK // 16)
        pltpu.sync_copy(o0, out_hbm.at[pl.ds(r0, _CHUNK), :])

    @pl.when(wid == jnp.int32(_EXTRA))
    def _():
        r0 = jnp.int32(_NFULL * _CHUNK)
        pltpu.sync_copy(feats_hbm.at[pl.ds(r0, _TAIL), :],
                        f0.at[pl.ds(jnp.int32(0), _TAIL), :])
        pltpu.sync_copy(velt_hbm.at[:, pl.ds(r0, _VTAIL)],
                        v0.at[:, pl.ds(jnp.int32(0), _VTAIL)])
        compute(f0, v0, o0, _TAIL // 16)
        pltpu.sync_copy(o0.at[pl.ds(jnp.int32(0), _TAIL), :],
                        out_hbm.at[pl.ds(r0, _TAIL), :])


def kernel(feats, vel_xy, coords):
    del coords
    n, c = feats.shape
    feats = feats.astype(jnp.float32)
    velt = vel_xy.astype(jnp.float32).T
    velt = jnp.pad(velt, ((0, 0), (0, _NFULL * _CHUNK + _VTAIL - n)))
    mesh = plsc.VectorSubcoreMesh(core_axis_name="c", subcore_axis_name="s")
    k = functools.partial(
        pl.kernel,
        mesh=mesh,
        out_type=jax.ShapeDtypeStruct((n, c), jnp.float32),
        scratch_types=[
            pltpu.VMEM((_CHUNK, _C), jnp.float32),
            pltpu.VMEM((_CHUNK, _C), jnp.float32),
            pltpu.VMEM((_CHUNK, _C), jnp.float32),
            pltpu.VMEM((_CHUNK, _C), jnp.float32),
            pltpu.VMEM((2, _VPAD), jnp.float32),
            pltpu.VMEM((2, _VPAD), jnp.float32),
            pltpu.SemaphoreType.DMA,
            pltpu.SemaphoreType.DMA,
            pltpu.SemaphoreType.DMA,
            pltpu.SemaphoreType.DMA,
        ],
    )(_sc_body)
    return k(feats, velt)
